# baseline (speedup 1.0000x reference)
import functools

import jax
import jax.numpy as jnp
from jax import lax
from jax.experimental import pallas as pl
from jax.experimental.pallas import tpu as pltpu
from jax.experimental.pallas import tpu_sc as plsc

N = 10000
E = 160000
DF = 256
H = 512
C = 128

NPAD = 10240
RPT = NPAD // 32
ACCR = RPT + 8
ECH = 2000
LCAP = 161280
FB = 1024
GC = 80
IB = 960
SPB = IB // GC

_ONES16 = None


def _filter_body(src_hbm, dst_hbm, lsrc_hbm, ldst_hbm, counts_hbm, degb_hbm,
                 schunk0, dchunk0, schunk1, dchunk1, lsv, ldv, hist, degv,
                 cntv, sems0, semd0, sems1, semd1):
    c = lax.axis_index("c")
    s = lax.axis_index("s")
    w = c * 16 + s
    base = w * RPT
    zi16 = jnp.zeros((16,), jnp.float32)
    ones16 = jnp.ones((16,), jnp.float32)

    def zh(i, _):
        hist[pl.ds(i * 16, 16)] = zi16
        return _
    lax.fori_loop(0, (RPT + 16) // 16, zh, None)

    def load(ch, sc, dc, ss, sd):
        pltpu.async_copy(src_hbm.at[pl.ds(ch * ECH, ECH)], sc, ss)
        pltpu.async_copy(dst_hbm.at[pl.ds(ch * ECH, ECH)], dc, sd)

    def drain(sc, dc, ss, sd):
        pltpu.make_async_copy(src_hbm.at[pl.ds(0, ECH)], sc, ss).wait()
        pltpu.make_async_copy(dst_hbm.at[pl.ds(0, ECH)], dc, sd).wait()

    load(0, schunk0, dchunk0, sems0, semd0)

    def chunk(ch, carry):
        nxt = ch + 1

        @pl.when((nxt < E // ECH) & (ch % 2 == 0))
        def _():
            load(nxt, schunk1, dchunk1, sems1, semd1)

        @pl.when((nxt < E // ECH) & (ch % 2 == 1))
        def _():
            load(nxt, schunk0, dchunk0, sems0, semd0)

        def proc(schunk, dchunk, ss, sd):
            drain(schunk, dchunk, ss, sd)

            def vec(v, carry2):
                cnt, hoff = carry2
                d16 = dchunk[pl.ds(v * 16, 16)]
                s16 = schunk[pl.ds(v * 16, 16)]
                dl = d16 - base
                own = (dl >= 0) & (dl < RPT)
                dlc = jnp.where(own, dl, RPT)
                plsc.addupdate_scatter(hist, [dlc], ones16, mask=own)
                plsc.store_compressed(lsv.at[pl.ds(cnt, 16)], s16, mask=own)
                plsc.store_compressed(ldv.at[pl.ds(cnt, 16)], dlc, mask=own)
                pcv = plsc.all_reduce_population_count(own)
                cnt = cnt + (pcv[0] if pcv.ndim else pcv)
                flush = cnt >= FB

                @pl.when(flush)
                def _():
                    ho = pl.multiple_of(w * LCAP + hoff, FB)
                    pltpu.sync_copy(lsv.at[pl.ds(0, FB)],
                                    lsrc_hbm.at[pl.ds(ho, FB)])
                    pltpu.sync_copy(ldv.at[pl.ds(0, FB)],
                                    ldst_hbm.at[pl.ds(ho, FB)])
                    rs = lsv[pl.ds(FB, 16)]
                    rd = ldv[pl.ds(FB, 16)]
                    lsv[pl.ds(0, 16)] = rs
                    ldv[pl.ds(0, 16)] = rd

                cnt = jnp.where(flush, cnt - FB, cnt)
                hoff = jnp.where(flush, hoff + FB, hoff)
                return (cnt, hoff)

            return lax.fori_loop(0, ECH // 16, vec, carry)

        return lax.cond(ch % 2 == 0,
                        lambda: proc(schunk0, dchunk0, sems0, semd0),
                        lambda: proc(schunk1, dchunk1, sems1, semd1))

    cnt, hoff = lax.fori_loop(0, E // ECH, chunk,
                              (jnp.int32(0), jnp.int32(0)))

    sent_s = jnp.zeros((16,), jnp.int32)
    sent_d = jnp.full((16,), RPT, jnp.int32)
    for j in range(8):
        lsv[pl.ds(cnt + 16 * j, 16)] = sent_s
        ldv[pl.ds(cnt + 16 * j, 16)] = sent_d
    ho = pl.multiple_of(w * LCAP + hoff, FB)
    pltpu.sync_copy(lsv.at[pl.ds(0, FB + 128)],
                    lsrc_hbm.at[pl.ds(ho, FB + 128)])
    pltpu.sync_copy(ldv.at[pl.ds(0, FB + 128)],
                    ldst_hbm.at[pl.ds(ho, FB + 128)])
    total = cnt + hoff
    cntv[pl.ds(0, 16)] = jnp.full((16,), total, jnp.int32)
    pltpu.sync_copy(cntv, counts_hbm.at[pl.ds(w * 16, 16)])

    def degm(m, _):
        h16 = hist[pl.ds(m * 16, 16)]
        for i in range(16):
            vv = jnp.full((16,), h16[i], jnp.float32)
            for k in range(16):
                degv[m * 16 + i, pl.ds(k * 16, 16)] = vv
        return _
    lax.fori_loop(0, RPT // 16, degm, None)
    pltpu.sync_copy(degv, degb_hbm.at[pl.ds(pl.multiple_of(base, RPT), RPT)])


def _agg_body(g_hbm, lsrc_hbm, ldst_hbm, counts_hbm, s_hbm,
              sidx, didx, rows0, rows1, acc, cntv, sem0, sem1):
    c = lax.axis_index("c")
    s = lax.axis_index("s")
    w = c * 16 + s
    zi16 = jnp.zeros((16,), jnp.float32)

    def za(r, _):
        for k in range(16):
            acc[r, pl.ds(k * 16, 16)] = zi16
        return _
    lax.fori_loop(0, ACCR, za, None)

    pltpu.sync_copy(counts_hbm.at[pl.ds(w * 16, 16)], cntv)
    total = cntv[pl.ds(0, 16)][0]
    nch = (total + (GC - 1)) // GC
    nblk = (total + (IB - 1)) // IB

    def adds(rows, dls_all, sub):
        def grp(m, __):
            dls = dls_all[pl.ds(sub * GC + m * 16, 16)]
            j0 = m * 16
            for i in range(16):
                dl = dls[i]
                for k in range(16):
                    plsc.addupdate(acc.at[dl, pl.ds(k * 16, 16)],
                                   rows[j0 + i, pl.ds(k * 16, 16)])
            return __
        lax.fori_loop(0, GC // 16, grp, None)

    def gather(rows, sem, blk, sub):
        pltpu.async_copy(g_hbm.at[sidx.at[pl.ds(sub * GC, GC)]], rows, sem)

    def drain(rows, sem):
        pltpu.make_async_copy(g_hbm.at[pl.ds(0, GC)], rows, sem).wait()

    def load_blk(blk):
        off = pl.multiple_of(w * LCAP + blk * IB, IB)
        pltpu.sync_copy(lsrc_hbm.at[pl.ds(off, IB)], sidx)
        pltpu.sync_copy(ldst_hbm.at[pl.ds(off, IB)], didx)

    load_blk(0)

    @pl.when(nch > 0)
    def _():
        gather(rows0, sem0, 0, 0)

    def chunk(ch, _):
        blk_next, sub_next = (ch + 1) // SPB, (ch + 1) % SPB

        @pl.when((ch + 1 < nch) & (sub_next != 0))
        def _():
            @pl.when(ch % 2 == 0)
            def _():
                gather(rows1, sem1, blk_next, sub_next)
            @pl.when(ch % 2 == 1)
            def _():
                gather(rows0, sem0, blk_next, sub_next)

        sub = ch % SPB

        @pl.when(ch % 2 == 0)
        def _():
            drain(rows0, sem0)
            adds(rows0, didx, sub)
        @pl.when(ch % 2 == 1)
        def _():
            drain(rows1, sem1)
            adds(rows1, didx, sub)

        @pl.when((ch + 1 < nch) & (sub_next == 0))
        def _():
            load_blk(blk_next)
            @pl.when(ch % 2 == 0)
            def _():
                gather(rows1, sem1, blk_next, 0)
            @pl.when(ch % 2 == 1)
            def _():
                gather(rows0, sem0, blk_next, 0)
        return _

    lax.fori_loop(0, nch, chunk, None)
    pltpu.sync_copy(acc.at[pl.ds(0, RPT)],
                    s_hbm.at[pl.ds(pl.multiple_of(w * RPT, RPT), RPT)])


@functools.lru_cache(maxsize=None)
def _sc_kernels():
    mesh = plsc.VectorSubcoreMesh(core_axis_name="c", subcore_axis_name="s",
                                  num_cores=2, num_subcores=16)
    params = pltpu.CompilerParams(needs_layout_passes=False)
    i32, f32 = jnp.int32, jnp.float32
    filt = pl.kernel(
        _filter_body,
        out_type=(jax.ShapeDtypeStruct((32 * LCAP,), i32),
                  jax.ShapeDtypeStruct((32 * LCAP,), i32),
                  jax.ShapeDtypeStruct((32 * 16,), i32),
                  jax.ShapeDtypeStruct((NPAD, DF), f32)),
        mesh=mesh,
        compiler_params=params,
        scratch_types=[
            pltpu.VMEM((ECH,), i32),
            pltpu.VMEM((ECH,), i32),
            pltpu.VMEM((ECH,), i32),
            pltpu.VMEM((ECH,), i32),
            pltpu.VMEM((FB + 256,), i32),
            pltpu.VMEM((FB + 256,), i32),
            pltpu.VMEM((RPT + 16,), f32),
            pltpu.VMEM((RPT, DF), f32),
            pltpu.VMEM((16,), i32),
            pltpu.SemaphoreType.DMA,
            pltpu.SemaphoreType.DMA,
            pltpu.SemaphoreType.DMA,
            pltpu.SemaphoreType.DMA,
        ],
    )
    agg = pl.kernel(
        _agg_body,
        out_type=jax.ShapeDtypeStruct((NPAD, DF), f32),
        mesh=mesh,
        compiler_params=params,
        scratch_types=[
            pltpu.VMEM((IB,), i32),
            pltpu.VMEM((IB,), i32),
            pltpu.VMEM((GC, DF), f32),
            pltpu.VMEM((GC, DF), f32),
            pltpu.VMEM((ACCR, DF), f32),
            pltpu.VMEM((16,), i32),
            pltpu.SemaphoreType.DMA,
            pltpu.SemaphoreType.DMA,
        ],
    )
    return filt, agg


RB = 512


def _k1_body(degb_ref, x_ref, gx_ref, dinv_ref):
    dinv = lax.rsqrt(degb_ref[...] + 1.0)
    gx_ref[...] = x_ref[...] * dinv
    dinv_ref[...] = dinv


def _k2_body(s_ref, gx_ref, dinv_ref, w1_ref, b1_ref, g1a_ref, g1b_ref):
    dv = dinv_ref[...]
    z = dv * (s_ref[...] + gx_ref[...])
    h1 = jnp.tanh(jnp.dot(z, w1_ref[...], preferred_element_type=jnp.float32)
                  + b1_ref[0:1, :])
    g1a_ref[...] = dv * h1[:, :DF]
    g1b_ref[...] = dv * h1[:, DF:]


def _k3_body(sa_ref, sb_ref, g1a_ref, g1b_ref, dinv_ref,
             w2_ref, b2_ref, wc_ref, bc_ref, h2_ref, out_ref):
    dv = dinv_ref[...]
    za = dv * (sa_ref[...] + g1a_ref[...])
    zb = dv * (sb_ref[...] + g1b_ref[...])
    z = jnp.concatenate([za, zb], axis=1)
    h2 = jnp.tanh(jnp.dot(z, w2_ref[...], preferred_element_type=jnp.float32)
                  + b2_ref[0:1, :])
    h2_ref[...] = h2
    out_ref[...] = (jnp.dot(h2, wc_ref[...], preferred_element_type=jnp.float32)
                    + bc_ref[0:1, :])


def _row_spec(cols):
    return pl.BlockSpec((RB, cols), lambda i: (i, 0))


def _full_spec(shape):
    return pl.BlockSpec(shape, lambda i: tuple(0 for _ in shape))


def kernel(x, edge_index, W1, b1, W2, b2, Wc, bc):
    f32 = jnp.float32
    src = edge_index[0]
    dst = edge_index[1]
    xp = jnp.concatenate([x, jnp.zeros((NPAD - N, DF), f32)], axis=0)
    _filter, _agg = _sc_kernels()

    lsrc, ldst, counts, degb = _filter(src, dst)

    gx, dinv = pl.pallas_call(
        _k1_body,
        grid=(NPAD // RB,),
        in_specs=[_row_spec(DF), _row_spec(DF)],
        out_specs=[_row_spec(DF), _row_spec(DF)],
        out_shape=[jax.ShapeDtypeStruct((NPAD, DF), f32),
                   jax.ShapeDtypeStruct((NPAD, DF), f32)],
    )(degb, xp)

    Sx = _agg(gx, lsrc, ldst, counts)

    b1t = jnp.broadcast_to(b1[None, :], (8, H))
    g1a, g1b = pl.pallas_call(
        _k2_body,
        grid=(NPAD // RB,),
        in_specs=[_row_spec(DF), _row_spec(DF), _row_spec(DF),
                  _full_spec((DF, H)), _full_spec((8, H))],
        out_specs=[_row_spec(DF), _row_spec(DF)],
        out_shape=[jax.ShapeDtypeStruct((NPAD, DF), f32),
                   jax.ShapeDtypeStruct((NPAD, DF), f32)],
    )(Sx, gx, dinv, W1, b1t)

    S2a = _agg(g1a, lsrc, ldst, counts)
    S2b = _agg(g1b, lsrc, ldst, counts)

    b2t = jnp.broadcast_to(b2[None, :], (8, H))
    bct = jnp.broadcast_to(bc[None, :], (8, C))
    h2, out = pl.pallas_call(
        _k3_body,
        grid=(NPAD // RB,),
        in_specs=[_row_spec(DF), _row_spec(DF), _row_spec(DF), _row_spec(DF),
                  _row_spec(DF), _full_spec((H, H)), _full_spec((8, H)),
                  _full_spec((H, C)), _full_spec((8, C))],
        out_specs=[_row_spec(H), _row_spec(C)],
        out_shape=[jax.ShapeDtypeStruct((NPAD, H), f32),
                   jax.ShapeDtypeStruct((NPAD, C), f32)],
    )(S2a, S2b, g1a, g1b, dinv, W2, b2t, Wc, bct)

    return (out[:N], h2[:N])


# speedup vs baseline: 1.0508x; 1.0508x over previous
import functools

import jax
import jax.numpy as jnp
from jax import lax
from jax.experimental import pallas as pl
from jax.experimental.pallas import tpu as pltpu
from jax.experimental.pallas import tpu_sc as plsc

N = 10000
E = 160000
DF = 256
H = 512
C = 128

NPAD = 10240
RPT = NPAD // 32
ACCR = RPT + 8
ECH = 2000
LCAP = 161280
FB = 1024
GC = 64
IB = 1024
SPB = IB // GC


def _filter_body(src_hbm, dst_hbm, lsrc_hbm, ldst_hbm, counts_hbm, degb_hbm,
                 schunk0, dchunk0, schunk1, dchunk1, lsv, ldv, hist, degv,
                 cntv, sems0, semd0, sems1, semd1):
    c = lax.axis_index("c")
    s = lax.axis_index("s")
    w = c * 16 + s
    base = w * RPT
    zi16 = jnp.zeros((16,), jnp.float32)
    ones16 = jnp.ones((16,), jnp.float32)

    def zh(i, _):
        hist[pl.ds(i * 16, 16)] = zi16
        return _
    lax.fori_loop(0, (RPT + 16) // 16, zh, None)

    def load(ch, sc, dc, ss, sd):
        pltpu.async_copy(src_hbm.at[pl.ds(ch * ECH, ECH)], sc, ss)
        pltpu.async_copy(dst_hbm.at[pl.ds(ch * ECH, ECH)], dc, sd)

    def drain(sc, dc, ss, sd):
        pltpu.make_async_copy(src_hbm.at[pl.ds(0, ECH)], sc, ss).wait()
        pltpu.make_async_copy(dst_hbm.at[pl.ds(0, ECH)], dc, sd).wait()

    load(0, schunk0, dchunk0, sems0, semd0)

    def chunk(ch, carry):
        nxt = ch + 1

        @pl.when((nxt < E // ECH) & (ch % 2 == 0))
        def _():
            load(nxt, schunk1, dchunk1, sems1, semd1)

        @pl.when((nxt < E // ECH) & (ch % 2 == 1))
        def _():
            load(nxt, schunk0, dchunk0, sems0, semd0)

        def proc(schunk, dchunk, ss, sd):
            drain(schunk, dchunk, ss, sd)

            def vec(v, carry2):
                cnt, hoff = carry2
                d16 = dchunk[pl.ds(v * 16, 16)]
                s16 = schunk[pl.ds(v * 16, 16)]
                dl = d16 - base
                own = (dl >= 0) & (dl < RPT)
                dlc = jnp.where(own, dl, RPT)
                plsc.addupdate_scatter(hist, [dlc], ones16, mask=own)
                plsc.store_compressed(lsv.at[pl.ds(cnt, 16)], s16, mask=own)
                plsc.store_compressed(ldv.at[pl.ds(cnt, 16)], dlc, mask=own)
                pcv = plsc.all_reduce_population_count(own)
                cnt = cnt + (pcv[0] if pcv.ndim else pcv)
                flush = cnt >= FB

                @pl.when(flush)
                def _():
                    ho = pl.multiple_of(w * LCAP + hoff, FB)
                    pltpu.sync_copy(lsv.at[pl.ds(0, FB)],
                                    lsrc_hbm.at[pl.ds(ho, FB)])
                    pltpu.sync_copy(ldv.at[pl.ds(0, FB)],
                                    ldst_hbm.at[pl.ds(ho, FB)])
                    rs = lsv[pl.ds(FB, 16)]
                    rd = ldv[pl.ds(FB, 16)]
                    lsv[pl.ds(0, 16)] = rs
                    ldv[pl.ds(0, 16)] = rd

                cnt = jnp.where(flush, cnt - FB, cnt)
                hoff = jnp.where(flush, hoff + FB, hoff)
                return (cnt, hoff)

            return lax.fori_loop(0, ECH // 16, vec, carry)

        return lax.cond(ch % 2 == 0,
                        lambda: proc(schunk0, dchunk0, sems0, semd0),
                        lambda: proc(schunk1, dchunk1, sems1, semd1))

    cnt, hoff = lax.fori_loop(0, E // ECH, chunk,
                              (jnp.int32(0), jnp.int32(0)))

    sent_s = jnp.zeros((16,), jnp.int32)
    sent_d = jnp.full((16,), RPT, jnp.int32)
    for j in range(8):
        lsv[pl.ds(cnt + 16 * j, 16)] = sent_s
        ldv[pl.ds(cnt + 16 * j, 16)] = sent_d
    ho = pl.multiple_of(w * LCAP + hoff, FB)
    pltpu.sync_copy(lsv.at[pl.ds(0, FB + 128)],
                    lsrc_hbm.at[pl.ds(ho, FB + 128)])
    pltpu.sync_copy(ldv.at[pl.ds(0, FB + 128)],
                    ldst_hbm.at[pl.ds(ho, FB + 128)])
    total = cnt + hoff
    cntv[pl.ds(0, 16)] = jnp.full((16,), total, jnp.int32)
    pltpu.sync_copy(cntv, counts_hbm.at[pl.ds(w * 16, 16)])

    def degm(m, _):
        h16 = hist[pl.ds(m * 16, 16)]
        for i in range(16):
            vv = jnp.full((16,), h16[i], jnp.float32)
            for k in range(16):
                degv[m * 16 + i, pl.ds(k * 16, 16)] = vv
        return _
    lax.fori_loop(0, RPT // 16, degm, None)
    pltpu.sync_copy(degv, degb_hbm.at[pl.ds(pl.multiple_of(base, RPT), RPT)])


def _agg_body(g_hbm, lsrc_hbm, ldst_hbm, counts_hbm, s_hbm,
              sidx, didx, rows0, rows1, acc, cntv, sem0, sem1):
    c = lax.axis_index("c")
    s = lax.axis_index("s")
    w = c * 16 + s
    zi16 = jnp.zeros((16,), jnp.float32)

    def za(r, _):
        for k in range(16):
            acc[r, pl.ds(k * 16, 16)] = zi16
        return _
    lax.fori_loop(0, ACCR, za, None)

    pltpu.sync_copy(counts_hbm.at[pl.ds(w * 16, 16)], cntv)
    total = cntv[pl.ds(0, 16)][0]
    nch = (total + (GC - 1)) // GC
    nblk = (total + (IB - 1)) // IB

    def adds(rows, dls_all, sub):
        def grp(m, __):
            dls = dls_all[pl.ds(sub * GC + m * 16, 16)]
            j0 = m * 16
            for i in range(16):
                dl = dls[i]
                for k in range(16):
                    plsc.addupdate(acc.at[dl, pl.ds(k * 16, 16)],
                                   rows[j0 + i, pl.ds(k * 16, 16)])
            return __
        lax.fori_loop(0, GC // 16, grp, None)

    def gather(rows, sem, blk, sub):
        pltpu.async_copy(g_hbm.at[sidx.at[pl.ds(sub * GC, GC)]], rows, sem)

    def drain(rows, sem):
        pltpu.make_async_copy(g_hbm.at[pl.ds(0, GC)], rows, sem).wait()

    def load_blk(blk):
        off = pl.multiple_of(w * LCAP + blk * IB, IB)
        pltpu.sync_copy(lsrc_hbm.at[pl.ds(off, IB)], sidx)
        pltpu.sync_copy(ldst_hbm.at[pl.ds(off, IB)], didx)

    load_blk(0)

    @pl.when(nch > 0)
    def _():
        gather(rows0, sem0, 0, 0)

    def chunk(ch, _):
        blk_next, sub_next = (ch + 1) // SPB, (ch + 1) % SPB

        @pl.when((ch + 1 < nch) & (sub_next != 0))
        def _():
            @pl.when(ch % 2 == 0)
            def _():
                gather(rows1, sem1, blk_next, sub_next)
            @pl.when(ch % 2 == 1)
            def _():
                gather(rows0, sem0, blk_next, sub_next)

        sub = ch % SPB

        @pl.when(ch % 2 == 0)
        def _():
            drain(rows0, sem0)
            adds(rows0, didx, sub)
        @pl.when(ch % 2 == 1)
        def _():
            drain(rows1, sem1)
            adds(rows1, didx, sub)

        @pl.when((ch + 1 < nch) & (sub_next == 0))
        def _():
            load_blk(blk_next)
            @pl.when(ch % 2 == 0)
            def _():
                gather(rows1, sem1, blk_next, 0)
            @pl.when(ch % 2 == 1)
            def _():
                gather(rows0, sem0, blk_next, 0)
        return _

    lax.fori_loop(0, nch, chunk, None)
    pltpu.sync_copy(acc.at[pl.ds(0, RPT)],
                    s_hbm.at[pl.ds(pl.multiple_of(w * RPT, RPT), RPT)])


@functools.lru_cache(maxsize=None)
def _sc_kernels():
    mesh = plsc.VectorSubcoreMesh(core_axis_name="c", subcore_axis_name="s",
                                  num_cores=2, num_subcores=16)
    params = pltpu.CompilerParams(needs_layout_passes=False)
    i32, f32 = jnp.int32, jnp.float32
    filt = pl.kernel(
        _filter_body,
        out_type=(jax.ShapeDtypeStruct((32 * LCAP,), i32),
                  jax.ShapeDtypeStruct((32 * LCAP,), i32),
                  jax.ShapeDtypeStruct((32 * 16,), i32),
                  jax.ShapeDtypeStruct((NPAD, DF), f32)),
        mesh=mesh,
        compiler_params=params,
        scratch_types=[
            pltpu.VMEM((ECH,), i32),
            pltpu.VMEM((ECH,), i32),
            pltpu.VMEM((ECH,), i32),
            pltpu.VMEM((ECH,), i32),
            pltpu.VMEM((FB + 256,), i32),
            pltpu.VMEM((FB + 256,), i32),
            pltpu.VMEM((RPT + 16,), f32),
            pltpu.VMEM((RPT, DF), f32),
            pltpu.VMEM((16,), i32),
            pltpu.SemaphoreType.DMA,
            pltpu.SemaphoreType.DMA,
            pltpu.SemaphoreType.DMA,
            pltpu.SemaphoreType.DMA,
        ],
    )
    agg = pl.kernel(
        _agg_body,
        out_type=jax.ShapeDtypeStruct((NPAD, DF), f32),
        mesh=mesh,
        compiler_params=params,
        scratch_types=[
            pltpu.VMEM((IB,), i32),
            pltpu.VMEM((IB,), i32),
            pltpu.VMEM((GC, DF), f32),
            pltpu.VMEM((GC, DF), f32),
            pltpu.VMEM((ACCR, DF), f32),
            pltpu.VMEM((16,), i32),
            pltpu.SemaphoreType.DMA,
            pltpu.SemaphoreType.DMA,
        ],
    )
    return filt, agg


RB = 512


def _k1_body(degb_ref, x_ref, gx_ref, dinv_ref):
    dinv = lax.rsqrt(degb_ref[...] + 1.0)
    gx_ref[...] = x_ref[...] * dinv
    dinv_ref[...] = dinv


def _k2_body(s_ref, gx_ref, dinv_ref, w1_ref, b1_ref, g1a_ref, g1b_ref):
    dv = dinv_ref[...]
    z = dv * (s_ref[...] + gx_ref[...])
    h1 = jnp.tanh(jnp.dot(z, w1_ref[...], preferred_element_type=jnp.float32)
                  + b1_ref[0:1, :])
    g1a_ref[...] = dv * h1[:, :DF]
    g1b_ref[...] = dv * h1[:, DF:]


def _k3_body(sa_ref, sb_ref, g1a_ref, g1b_ref, dinv_ref,
             w2_ref, b2_ref, wc_ref, bc_ref, h2_ref, out_ref):
    dv = dinv_ref[...]
    za = dv * (sa_ref[...] + g1a_ref[...])
    zb = dv * (sb_ref[...] + g1b_ref[...])
    z = jnp.concatenate([za, zb], axis=1)
    h2 = jnp.tanh(jnp.dot(z, w2_ref[...], preferred_element_type=jnp.float32)
                  + b2_ref[0:1, :])
    h2_ref[...] = h2
    out_ref[...] = (jnp.dot(h2, wc_ref[...], preferred_element_type=jnp.float32)
                    + bc_ref[0:1, :])


def _row_spec(cols):
    return pl.BlockSpec((RB, cols), lambda i: (i, 0))


def _full_spec(shape):
    return pl.BlockSpec(shape, lambda i: tuple(0 for _ in shape))


def kernel(x, edge_index, W1, b1, W2, b2, Wc, bc):
    f32 = jnp.float32
    src = edge_index[0]
    dst = edge_index[1]
    xp = jnp.concatenate([x, jnp.zeros((NPAD - N, DF), f32)], axis=0)
    _filter, _agg = _sc_kernels()

    lsrc, ldst, counts, degb = _filter(src, dst)

    gx, dinv = pl.pallas_call(
        _k1_body,
        grid=(NPAD // RB,),
        in_specs=[_row_spec(DF), _row_spec(DF)],
        out_specs=[_row_spec(DF), _row_spec(DF)],
        out_shape=[jax.ShapeDtypeStruct((NPAD, DF), f32),
                   jax.ShapeDtypeStruct((NPAD, DF), f32)],
    )(degb, xp)

    Sx = _agg(gx, lsrc, ldst, counts)

    b1t = jnp.broadcast_to(b1[None, :], (8, H))
    g1a, g1b = pl.pallas_call(
        _k2_body,
        grid=(NPAD // RB,),
        in_specs=[_row_spec(DF), _row_spec(DF), _row_spec(DF),
                  _full_spec((DF, H)), _full_spec((8, H))],
        out_specs=[_row_spec(DF), _row_spec(DF)],
        out_shape=[jax.ShapeDtypeStruct((NPAD, DF), f32),
                   jax.ShapeDtypeStruct((NPAD, DF), f32)],
    )(Sx, gx, dinv, W1, b1t)

    S2a = _agg(g1a, lsrc, ldst, counts)
    S2b = _agg(g1b, lsrc, ldst, counts)

    b2t = jnp.broadcast_to(b2[None, :], (8, H))
    bct = jnp.broadcast_to(bc[None, :], (8, C))
    h2, out = pl.pallas_call(
        _k3_body,
        grid=(NPAD // RB,),
        in_specs=[_row_spec(DF), _row_spec(DF), _row_spec(DF), _row_spec(DF),
                  _row_spec(DF), _full_spec((H, H)), _full_spec((8, H)),
                  _full_spec((H, C)), _full_spec((8, C))],
        out_specs=[_row_spec(H), _row_spec(C)],
        out_shape=[jax.ShapeDtypeStruct((NPAD, H), f32),
                   jax.ShapeDtypeStruct((NPAD, C), f32)],
    )(S2a, S2b, g1a, g1b, dinv, W2, b2t, Wc, bct)

    return (out[:N], h2[:N])
